# Initial kernel scaffold; baseline (speedup 1.0000x reference)
import jax
import jax.numpy as jnp
from jax.experimental import pallas as pl


def kernel(x, W, b):
    raise NotImplementedError("write your pallas kernel here")


# speedup vs baseline: 1.0186x; 1.0186x over previous
import functools

import jax
import jax.numpy as jnp
from jax import lax
from jax.experimental import pallas as pl
from jax.experimental.pallas import tpu as pltpu
from jax.experimental.pallas import tpu_sc as plsc

KEEPK = 64
BB, TT, CC = 4, 2048, 2048
SUB, LANE = 16, 128
TBLK = 512


def _attn_body(x_ref, w_ref, b_ref, attn_ref):
    attn_ref[0] = lax.dot_general(
        x_ref[0], w_ref[...], (((1,), (0,)), ((), ())),
        preferred_element_type=jnp.float32) + b_ref[0, 0]


def _attn(x, w, b2):
    return pl.pallas_call(
        _attn_body,
        grid=(BB,),
        in_specs=[
            pl.BlockSpec((1, TT, CC), lambda b: (b, 0, 0)),
            pl.BlockSpec((CC, 1), lambda b: (0, 0)),
            pl.BlockSpec((1, 1), lambda b: (0, 0)),
        ],
        out_specs=pl.BlockSpec((1, TT, 1), lambda b: (b, 0, 0)),
        out_shape=jax.ShapeDtypeStruct((BB, TT, 1), jnp.float32),
    )(x, w, b2)


def _topk_body(attn_ref, idxg_ref, idxl_ref):
    bi = pl.program_id(0)
    attn = attn_ref[0]

    fi = (lax.broadcasted_iota(jnp.int32, (SUB, LANE), 0) * LANE
          + lax.broadcasted_iota(jnp.int32, (SUB, LANE), 1))
    lane_k = lax.broadcasted_iota(jnp.int32, (1, KEEPK), 1)
    sub_k = lax.broadcasted_iota(jnp.int32, (KEEPK, 1), 0)

    def body(i, carry):
        a, accg, accl = carry
        m = jnp.max(a)
        amin = jnp.min(jnp.where(a == m, fi, TT))
        a = jnp.where(fi == amin, -jnp.inf, a)
        accg = accg + jnp.where(lane_k == i, amin + bi * TT, 0)
        accl = accl + jnp.where(sub_k == i, amin, 0)
        return a, accg, accl

    _, accg, accl = lax.fori_loop(
        0, KEEPK, body,
        (attn,
         jnp.zeros((1, KEEPK), jnp.int32),
         jnp.zeros((KEEPK, 1), jnp.int32)))
    idxg_ref[0] = accg
    idxl_ref[0] = accl


def _topk(attn3):
    return pl.pallas_call(
        _topk_body,
        grid=(BB,),
        in_specs=[pl.BlockSpec((1, SUB, LANE), lambda b: (b, 0, 0))],
        out_specs=[
            pl.BlockSpec((1, 1, KEEPK), lambda b: (b, 0, 0)),
            pl.BlockSpec((1, KEEPK, 1), lambda b: (b, 0, 0)),
        ],
        out_shape=[
            jax.ShapeDtypeStruct((BB, 1, KEEPK), jnp.int32),
            jax.ShapeDtypeStruct((BB, KEEPK, 1), jnp.int32),
        ],
    )(attn3)


def _sc_gather(x2d, idx_flat):
    info = plsc.get_sparse_core_info()
    nw = info.num_cores * info.num_subcores
    nrows = BB * KEEPK
    bpw = nrows // nw
    mesh = plsc.VectorSubcoreMesh(core_axis_name="c", subcore_axis_name="s")

    @functools.partial(
        pl.kernel,
        mesh=mesh,
        out_type=jax.ShapeDtypeStruct((nrows, CC), jnp.float32),
        scratch_types=[
            pltpu.VMEM((bpw,), jnp.int32),
            pltpu.VMEM((bpw, CC), jnp.float32),
            pltpu.SemaphoreType.DMA,
        ],
    )
    def gk(x_hbm, idx_hbm, out_hbm, idx_v, rows_v, sem):
        wid = lax.axis_index("s") * info.num_cores + lax.axis_index("c")
        base = wid * bpw
        pltpu.sync_copy(idx_hbm.at[pl.ds(base, bpw)], idx_v)
        pltpu.async_copy(x_hbm.at[idx_v], rows_v, sem).wait()
        pltpu.sync_copy(rows_v, out_hbm.at[pl.ds(base, bpw)])

    return gk(x2d, idx_flat)


def _scatter_body(g_ref, il_ref, o_ref):
    il = il_ref[0]
    onehot = (il == lax.broadcasted_iota(jnp.int32, (KEEPK, CC), 1)
              ).astype(jnp.bfloat16)
    g = g_ref[...]
    hi32 = lax.bitcast_convert_type(
        lax.bitcast_convert_type(g, jnp.uint32) & jnp.uint32(0xFFFF0000),
        jnp.float32)
    hi = hi32.astype(jnp.bfloat16)
    lo = (g - hi32).astype(jnp.bfloat16)
    ghl = jnp.concatenate([hi, lo], axis=0)
    ohh = jnp.concatenate([onehot, onehot], axis=0)
    o_ref[0] = lax.dot_general(
        ghl, ohh, (((0,), (0,)), ((), ())),
        preferred_element_type=jnp.float32)


def _scatter(g, idx_l):
    return pl.pallas_call(
        _scatter_body,
        grid=(BB, TT // TBLK),
        in_specs=[
            pl.BlockSpec((KEEPK, TBLK), lambda b, t: (b, t)),
            pl.BlockSpec((1, KEEPK, 1), lambda b, t: (b, 0, 0)),
        ],
        out_specs=pl.BlockSpec((1, TBLK, CC), lambda b, t: (b, t, 0)),
        out_shape=jax.ShapeDtypeStruct((BB, TT, CC), jnp.float32),
    )(g, idx_l)


def kernel(x, W, b):
    attn = _attn(x, W, b.reshape(1, 1))
    idx_g, idx_l = _topk(attn.reshape(BB, SUB, LANE))
    g = _sc_gather(x.reshape(BB * TT, CC), idx_g.reshape(BB * KEEPK))
    return _scatter(g, idx_l)


# speedup vs baseline: 2.0966x; 2.0582x over previous
import functools

import jax
import jax.numpy as jnp
from jax import lax
from jax.experimental import pallas as pl
from jax.experimental.pallas import tpu as pltpu
from jax.experimental.pallas import tpu_sc as plsc

KEEPK = 64
BB, TT, CC = 4, 2048, 2048
TBLK = 512
RCH = 256


def _attn_sel_body(x_ref, w_ref, b_ref, s_ref, idxg_ref):
    bi = pl.program_id(0)
    xb = x_ref[0]
    a_col = lax.dot_general(
        xb, w_ref[...], (((1,), (0,)), ((), ())),
        preferred_element_type=jnp.float32) + b_ref[0, 0]
    a_row = lax.transpose(a_col, (1, 0))
    i_row = lax.broadcasted_iota(jnp.int32, (1, TT), 1)

    acc = jnp.zeros((RCH, TT), jnp.int32)
    for k in range(TT // RCH):
        ac = lax.slice(a_col, (k * RCH, 0), ((k + 1) * RCH, 1))
        ic = lax.broadcasted_iota(jnp.int32, (RCH, 1), 0) + k * RCH
        beats = (ac > a_row) | ((ac == a_row) & (ic < i_row))
        acc = acc + beats.astype(jnp.int32)
    rank = jnp.sum(acc, axis=0, keepdims=True)
    sel = rank < KEEPK
    m = sel.astype(jnp.int32)

    cum = m
    sh = 1
    while sh < TT:
        cum = cum + jnp.concatenate(
            [jnp.zeros((1, sh), jnp.int32), cum[:, :TT - sh]], axis=1)
        sh *= 2
    pos = cum - m

    sub_k = lax.broadcasted_iota(jnp.int32, (KEEPK, TT), 0)
    onehot = (pos == sub_k) & sel
    s_ref[0] = onehot.astype(jnp.bfloat16)
    gidx = jnp.where(onehot, i_row + bi * TT, 0)
    idxg_ref[0] = jnp.sum(gidx, axis=1, keepdims=True)


def _attn_sel(x, w, b2):
    return pl.pallas_call(
        _attn_sel_body,
        grid=(BB,),
        in_specs=[
            pl.BlockSpec((1, TT, CC), lambda b: (b, 0, 0)),
            pl.BlockSpec((CC, 1), lambda b: (0, 0)),
            pl.BlockSpec((1, 1), lambda b: (0, 0)),
        ],
        out_specs=[
            pl.BlockSpec((1, KEEPK, CC), lambda b: (b, 0, 0)),
            pl.BlockSpec((1, KEEPK, 1), lambda b: (b, 0, 0)),
        ],
        out_shape=[
            jax.ShapeDtypeStruct((BB, KEEPK, CC), jnp.bfloat16),
            jax.ShapeDtypeStruct((BB, KEEPK, 1), jnp.int32),
        ],
    )(x, w, b2)


def _sc_gather(x2d, idx_flat):
    info = plsc.get_sparse_core_info()
    nw = info.num_cores * info.num_subcores
    nrows = BB * KEEPK
    bpw = nrows // nw
    mesh = plsc.VectorSubcoreMesh(core_axis_name="c", subcore_axis_name="s")

    @functools.partial(
        pl.kernel,
        mesh=mesh,
        out_type=jax.ShapeDtypeStruct((nrows, CC), jnp.float32),
        scratch_types=[
            pltpu.VMEM((bpw,), jnp.int32),
            pltpu.VMEM((bpw, CC), jnp.float32),
            pltpu.SemaphoreType.DMA,
        ],
    )
    def gk(x_hbm, idx_hbm, out_hbm, idx_v, rows_v, sem):
        wid = lax.axis_index("s") * info.num_cores + lax.axis_index("c")
        base = wid * bpw
        pltpu.sync_copy(idx_hbm.at[pl.ds(base, bpw)], idx_v)
        pltpu.async_copy(x_hbm.at[idx_v], rows_v, sem).wait()
        pltpu.sync_copy(rows_v, out_hbm.at[pl.ds(base, bpw)])

    return gk(x2d, idx_flat)


def _scatter_body(g_ref, s_ref, o_ref):
    onehot = s_ref[0]
    g = g_ref[...]
    hi32 = lax.bitcast_convert_type(
        lax.bitcast_convert_type(g, jnp.uint32) & jnp.uint32(0xFFFF0000),
        jnp.float32)
    hi = hi32.astype(jnp.bfloat16)
    lo = (g - hi32).astype(jnp.bfloat16)
    ghl = jnp.concatenate([hi, lo], axis=0)
    ohh = jnp.concatenate([onehot, onehot], axis=0)
    o_ref[0] = lax.dot_general(
        ghl, ohh, (((0,), (0,)), ((), ())),
        preferred_element_type=jnp.float32)


def _scatter(g, s):
    return pl.pallas_call(
        _scatter_body,
        grid=(BB, TT // TBLK),
        in_specs=[
            pl.BlockSpec((KEEPK, TBLK), lambda b, t: (b, t)),
            pl.BlockSpec((1, KEEPK, CC), lambda b, t: (b, 0, 0)),
        ],
        out_specs=pl.BlockSpec((1, TBLK, CC), lambda b, t: (b, t, 0)),
        out_shape=jax.ShapeDtypeStruct((BB, TT, CC), jnp.float32),
    )(g, s)


def kernel(x, W, b):
    s, idx_g = _attn_sel(x, W, b.reshape(1, 1))
    g = _sc_gather(x.reshape(BB * TT, CC), idx_g.reshape(BB * KEEPK))
    return _scatter(g, s)
